# baseline (speedup 1.0000x reference)
import functools

import jax
import jax.numpy as jnp
from jax import lax
from jax.experimental import pallas as pl
from jax.experimental.pallas import tpu as pltpu


def _pair_eca_kernel(rows_per_img, half, w_ref, x_ref, o_ref):
    x = x_ref[...]
    R, L = x.shape
    cut = (half // 128) * 128
    rem = half - cut

    if rem:
        mid_end = min(cut + 128, L)
        xa = x[:, :cut] if cut else None
        xm = x[:, cut:mid_end]
        xb = x[:, mid_end:] if mid_end < L else None
        lane = lax.broadcasted_iota(jnp.int32, (R, mid_end - cut), 1)
        in_a = lane < rem
        zero = jnp.zeros_like(xm)
        sa = jnp.sum(jnp.where(in_a, xm, zero), axis=-1, keepdims=True)
        sb = jnp.sum(jnp.where(in_a, zero, xm), axis=-1, keepdims=True)
        if xa is not None:
            sa = sa + jnp.sum(xa, axis=-1, keepdims=True)
        if xb is not None:
            sb = sb + jnp.sum(xb, axis=-1, keepdims=True)
    else:
        xa = x[:, :half]
        xb = x[:, half:]
        sa = jnp.sum(xa, axis=-1, keepdims=True)
        sb = jnp.sum(xb, axis=-1, keepdims=True)

    z = jnp.zeros((1, 1), dtype=sa.dtype)
    sb_up = jnp.concatenate([z, sb[:-1, :]], axis=0)
    sa_dn = jnp.concatenate([sa[1:, :], z], axis=0)
    if R != rows_per_img:
        row = lax.broadcasted_iota(jnp.int32, (R, 1), 0) % rows_per_img
        sb_up = jnp.where(row == 0, 0.0, sb_up)
        sa_dn = jnp.where(row == rows_per_img - 1, 0.0, sa_dn)

    w0, w1, w2 = w_ref[0], w_ref[1], w_ref[2]
    att_a = jax.nn.sigmoid(w0 * sb_up + w1 * sa + w2 * sb)
    att_b = jax.nn.sigmoid(w0 * sa + w1 * sb + w2 * sa_dn)

    if rem:
        pieces = []
        if xa is not None:
            pieces.append(xa * att_a)
        pieces.append(xm * jnp.where(in_a, att_a, att_b))
        if xb is not None:
            pieces.append(xb * att_b)
        o_ref[...] = jnp.concatenate(pieces, axis=1)
    else:
        o_ref[...] = jnp.concatenate([xa * att_a, xb * att_b], axis=1)


def kernel(x_nchw, conv_weight):
    B, C, H, W = x_nchw.shape
    HW = H * W
    assert C % 2 == 0, "kernel assumes an even channel count"
    dtype = x_nchw.dtype
    itemsize = jnp.dtype(dtype).itemsize

    w = conv_weight.reshape(-1).astype(jnp.float32)
    assert w.shape[0] == 3, "specialized for k_size=3"
    w = w * (1.0 / float(HW))

    rows_img = C // 2
    L = 2 * HW
    x2 = x_nchw.reshape(B * rows_img, L)

    budget = 10 * 1024 * 1024
    nb = 1
    for cand in range(B, 0, -1):
        if B % cand == 0 and cand * rows_img * L * itemsize <= budget and B // cand >= 2:
            nb = cand
            break
    rows = nb * rows_img

    grid = (B // nb,)
    block_bytes = rows * L * itemsize
    vmem_limit = int(min(48 * 1024 * 1024, max(32 * 1024 * 1024, 6 * block_bytes)))

    out2 = pl.pallas_call(
        functools.partial(_pair_eca_kernel, rows_img, HW),
        out_shape=jax.ShapeDtypeStruct((B * rows_img, L), dtype),
        grid=grid,
        in_specs=[
            pl.BlockSpec(memory_space=pltpu.SMEM),
            pl.BlockSpec((rows, L), lambda b: (b, 0)),
        ],
        out_specs=pl.BlockSpec((rows, L), lambda b: (b, 0)),
        compiler_params=pltpu.CompilerParams(
            dimension_semantics=("parallel",),
            vmem_limit_bytes=vmem_limit,
        ),
        cost_estimate=pl.CostEstimate(
            flops=int(2 * B * C * HW + 8 * B * C),
            transcendentals=int(B * C),
            bytes_accessed=int(2 * B * C * HW * itemsize),
        ),
    )(w, x2)
    return out2.reshape(B, C, H, W)


# speedup vs baseline: 1.4130x; 1.4130x over previous
import functools

import jax
import jax.numpy as jnp
from jax.experimental import pallas as pl
from jax.experimental.pallas import tpu as pltpu


def _eca4d_kernel(w_ref, x_ref, o_ref):
    x = x_ref[0]
    s = jnp.sum(x, axis=(1, 2), keepdims=True)
    z = jnp.zeros((1, 1, 1), dtype=s.dtype)
    s_up = jnp.concatenate([z, s[:-1]], axis=0)
    s_dn = jnp.concatenate([s[1:], z], axis=0)
    att = jax.nn.sigmoid(w_ref[0] * s_up + w_ref[1] * s + w_ref[2] * s_dn)
    o_ref[0] = x * att.astype(x.dtype)


def kernel(x_nchw, conv_weight):
    B, C, H, W = x_nchw.shape
    HW = H * W
    dtype = x_nchw.dtype
    itemsize = jnp.dtype(dtype).itemsize

    w = conv_weight.reshape(-1).astype(jnp.float32)
    assert w.shape[0] == 3, "specialized for k_size=3"
    w = w * (1.0 / float(HW))

    out = pl.pallas_call(
        _eca4d_kernel,
        out_shape=jax.ShapeDtypeStruct((B, C, H, W), dtype),
        grid=(B,),
        in_specs=[
            pl.BlockSpec(memory_space=pltpu.SMEM),
            pl.BlockSpec((1, C, H, W), lambda b: (b, 0, 0, 0)),
        ],
        out_specs=pl.BlockSpec((1, C, H, W), lambda b: (b, 0, 0, 0)),
        compiler_params=pltpu.CompilerParams(
            dimension_semantics=("parallel",),
            vmem_limit_bytes=48 * 1024 * 1024,
        ),
        cost_estimate=pl.CostEstimate(
            flops=int(2 * B * C * HW + 8 * B * C),
            transcendentals=int(B * C),
            bytes_accessed=int(2 * B * C * HW * itemsize),
        ),
    )(w, x_nchw)
    return out


# speedup vs baseline: 9.4620x; 6.6962x over previous
import jax
import jax.numpy as jnp
from jax.experimental import pallas as pl
from jax.experimental.pallas import tpu as pltpu


def _eca_nhwc_kernel(w_ref, x_ref, o_ref):
    x = x_ref[0]
    s = jnp.sum(x, axis=(0, 1), keepdims=True)
    z = jnp.zeros((1, 1, 1), dtype=s.dtype)
    s_lo = jnp.concatenate([z, s[:, :, :-1]], axis=2)
    s_hi = jnp.concatenate([s[:, :, 1:], z], axis=2)
    att = jax.nn.sigmoid(w_ref[0] * s_lo + w_ref[1] * s + w_ref[2] * s_hi)
    o_ref[0] = x * att.astype(x.dtype)


def kernel(x_nchw, conv_weight):
    B, C, H, W = x_nchw.shape
    HW = H * W
    dtype = x_nchw.dtype
    itemsize = jnp.dtype(dtype).itemsize

    w = conv_weight.reshape(-1).astype(jnp.float32)
    assert w.shape[0] == 3, "specialized for k_size=3"
    w = w * (1.0 / float(HW))

    x_bhwc = jnp.transpose(x_nchw, (0, 2, 3, 1))

    out_bhwc = pl.pallas_call(
        _eca_nhwc_kernel,
        out_shape=jax.ShapeDtypeStruct((B, H, W, C), dtype),
        grid=(B,),
        in_specs=[
            pl.BlockSpec(memory_space=pltpu.SMEM),
            pl.BlockSpec((1, H, W, C), lambda b: (b, 0, 0, 0)),
        ],
        out_specs=pl.BlockSpec((1, H, W, C), lambda b: (b, 0, 0, 0)),
        compiler_params=pltpu.CompilerParams(
            dimension_semantics=("parallel",),
            vmem_limit_bytes=48 * 1024 * 1024,
        ),
        cost_estimate=pl.CostEstimate(
            flops=int(2 * B * C * HW + 8 * B * C),
            transcendentals=int(B * C),
            bytes_accessed=int(2 * B * C * HW * itemsize),
        ),
    )(w, x_bhwc)
    return jnp.transpose(out_bhwc, (0, 3, 1, 2))


# speedup vs baseline: 10.0369x; 1.0608x over previous
import functools

import jax
import jax.numpy as jnp
from jax.experimental import pallas as pl
from jax.experimental.pallas import tpu as pltpu


def _eca_nhwc_kernel(nb, w_ref, x_ref, o_ref):
    for i in range(nb):
        x = x_ref[i]
        s = jnp.sum(x, axis=(0, 1), keepdims=True)
        z = jnp.zeros((1, 1, 1), dtype=s.dtype)
        s_lo = jnp.concatenate([z, s[:, :, :-1]], axis=2)
        s_hi = jnp.concatenate([s[:, :, 1:], z], axis=2)
        att = jax.nn.sigmoid(w_ref[0] * s_lo + w_ref[1] * s + w_ref[2] * s_hi)
        o_ref[i] = x * att.astype(x.dtype)


def kernel(x_nchw, conv_weight):
    B, C, H, W = x_nchw.shape
    HW = H * W
    dtype = x_nchw.dtype
    itemsize = jnp.dtype(dtype).itemsize

    w = conv_weight.reshape(-1).astype(jnp.float32)
    assert w.shape[0] == 3, "specialized for k_size=3"
    w = w * (1.0 / float(HW))

    x_bhwc = jnp.transpose(x_nchw, (0, 2, 3, 1))

    nb = 2 if B % 2 == 0 else 1

    out_bhwc = pl.pallas_call(
        functools.partial(_eca_nhwc_kernel, nb),
        out_shape=jax.ShapeDtypeStruct((B, H, W, C), dtype),
        grid=(B // nb,),
        in_specs=[
            pl.BlockSpec(memory_space=pltpu.SMEM),
            pl.BlockSpec((nb, H, W, C), lambda b: (b, 0, 0, 0)),
        ],
        out_specs=pl.BlockSpec((nb, H, W, C), lambda b: (b, 0, 0, 0)),
        compiler_params=pltpu.CompilerParams(
            dimension_semantics=("parallel",),
            vmem_limit_bytes=48 * 1024 * 1024,
        ),
        cost_estimate=pl.CostEstimate(
            flops=int(2 * B * C * HW + 8 * B * C),
            transcendentals=int(B * C),
            bytes_accessed=int(2 * B * C * HW * itemsize),
        ),
    )(w, x_bhwc)
    return jnp.transpose(out_bhwc, (0, 3, 1, 2))


# speedup vs baseline: 10.1357x; 1.0098x over previous
import functools

import jax
import jax.numpy as jnp
from jax.experimental import pallas as pl
from jax.experimental.pallas import tpu as pltpu


def _eca_nhwc_kernel(nb, w_ref, x_ref, o_ref):
    for i in range(nb):
        x = x_ref[i]
        s = jnp.sum(x, axis=(0, 1), keepdims=True)
        z = jnp.zeros((1, 1, 1), dtype=s.dtype)
        s_lo = jnp.concatenate([z, s[:, :, :-1]], axis=2)
        s_hi = jnp.concatenate([s[:, :, 1:], z], axis=2)
        att = jax.nn.sigmoid(w_ref[0] * s_lo + w_ref[1] * s + w_ref[2] * s_hi)
        o_ref[i] = x * att.astype(x.dtype)


def kernel(x_nchw, conv_weight):
    B, C, H, W = x_nchw.shape
    HW = H * W
    dtype = x_nchw.dtype
    itemsize = jnp.dtype(dtype).itemsize

    w = conv_weight.reshape(-1).astype(jnp.float32)
    assert w.shape[0] == 3, "specialized for k_size=3"
    w = w * (1.0 / float(HW))

    x_bhwc = jnp.transpose(x_nchw, (0, 2, 3, 1))

    nb = 4 if B % 4 == 0 else (2 if B % 2 == 0 else 1)

    out_bhwc = pl.pallas_call(
        functools.partial(_eca_nhwc_kernel, nb),
        out_shape=jax.ShapeDtypeStruct((B, H, W, C), dtype),
        grid=(B // nb,),
        in_specs=[
            pl.BlockSpec(memory_space=pltpu.SMEM),
            pl.BlockSpec((nb, H, W, C), lambda b: (b, 0, 0, 0)),
        ],
        out_specs=pl.BlockSpec((nb, H, W, C), lambda b: (b, 0, 0, 0)),
        compiler_params=pltpu.CompilerParams(
            dimension_semantics=("parallel",),
            vmem_limit_bytes=56 * 1024 * 1024,
        ),
        cost_estimate=pl.CostEstimate(
            flops=int(2 * B * C * HW + 8 * B * C),
            transcendentals=int(B * C),
            bytes_accessed=int(2 * B * C * HW * itemsize),
        ),
    )(w, x_bhwc)
    return jnp.transpose(out_bhwc, (0, 3, 1, 2))
